# baseline (speedup 1.0000x reference)
import functools

import jax
import jax.numpy as jnp
from jax import lax
from jax.experimental import pallas as pl
from jax.experimental.pallas import tpu as pltpu
from jax.experimental.pallas import tpu_sc as plsc

_D = 32
_LP = 56
_PK = 8
_BM = 128
_RPB = _BM * _LP // _PK

_NW = 32


def _transpose_pad_table(tab):
    tab_t = tab.T
    n = tab_t.shape[1]
    bw = 4096

    def body(x_ref, o_ref):
        y = x_ref[...].T.astype(jnp.bfloat16)
        o_ref[...] = jnp.concatenate(
            [y, jnp.zeros((y.shape[0], 96), jnp.bfloat16)], axis=1)

    return pl.pallas_call(
        body,
        grid=(pl.cdiv(n, bw),),
        in_specs=[pl.BlockSpec((32, bw), lambda i: (0, i))],
        out_specs=pl.BlockSpec((bw, 128), lambda i: (i, 0)),
        out_shape=jax.ShapeDtypeStruct((n, 128), jnp.bfloat16),
    )(tab_t)


def _sc_gather(hist2d4, v2e_rows):
    nrow = hist2d4.shape[0]
    rw = nrow // _NW
    d = 32
    mesh = plsc.VectorSubcoreMesh(core_axis_name="c", subcore_axis_name="s")
    kbuf = 8
    assert rw % kbuf == 0

    @functools.partial(
        pl.kernel,
        out_type=jax.ShapeDtypeStruct((nrow * 128, d), jnp.bfloat16),
        mesh=mesh,
        scratch_types=[
            pltpu.VMEM((rw, 128), jnp.int32),
            [pltpu.VMEM((128, d), jnp.bfloat16) for _ in range(kbuf)],
            [pltpu.SemaphoreType.DMA for _ in range(kbuf)],
            [pltpu.SemaphoreType.DMA for _ in range(kbuf)],
        ],
        compiler_params=pltpu.CompilerParams(use_tc_tiling_on_sc=False),
    )
    def k(hist_hbm, v2e_hbm, euv_out, idx_v, bufs, gsems, wsems):
        wid = lax.axis_index("s") * 2 + lax.axis_index("c")
        pltpu.sync_copy(hist_hbm.at[pl.ds(wid * rw, rw), :], idx_v)

        def group(g, carry):
            c0 = g * kbuf
            for b in range(kbuf):
                pltpu.async_copy(v2e_hbm.at[idx_v.at[c0 + b]], bufs[b],
                                 gsems[b])
            for b in range(kbuf):
                base = (wid * rw + c0 + b) * 128
                pltpu.make_async_copy(v2e_hbm.at[idx_v.at[c0 + b]], bufs[b],
                                      gsems[b]).wait()
                pltpu.async_copy(bufs[b], euv_out.at[pl.ds(base, 128), :],
                                 wsems[b])
            for b in range(kbuf):
                base = (wid * rw + c0 + b) * 128
                pltpu.make_async_copy(bufs[b], euv_out.at[pl.ds(base, 128), :],
                                      wsems[b]).wait()
            return carry

        lax.fori_loop(0, rw // kbuf, group, 0)

    return k(hist2d4, v2e_rows)


def _tc_body(ep_ref, hr_ref, uv_ref, w1_ref, w2_ref, wa1_ref, wa2_ref,
             w3_ref, b1_ref, b2_ref, ba1_ref, ba2_ref, r1rt_ref, t8_ref,
             tsum_ref, ttile_ref, wa1b_ref, rep7_ref, rep7t_ref, out_ref):
    f32 = jnp.float32
    dot = functools.partial(jnp.dot, preferred_element_type=f32)
    e = ep_ref[...].astype(f32)
    t8 = t8_ref[...]
    hre = dot(hr_ref[...], t8)
    r1rt = r1rt_ref[...]
    er = jnp.where(hre == 0., r1rt[0:1],
         jnp.where(hre == 1., r1rt[1:2],
         jnp.where(hre == 2., r1rt[2:3],
         jnp.where(hre == 3., r1rt[3:4], r1rt[4:5]))))
    x = jnp.maximum(dot(e, w1_ref[...]) + er + b1_ref[...], 0.)
    o = jnp.maximum(dot(x, w2_ref[...]) + b2_ref[...], 0.)
    uvp = dot(dot(uv_ref[...], wa1b_ref[...]), ttile_ref[...])
    uvr = dot(rep7_ref[...], uvp)
    a1 = jnp.maximum(dot(o, wa1_ref[...]) + uvr + ba1_ref[...], 0.)
    a2 = jnp.maximum(dot(a1, wa2_ref[...]) + ba2_ref[...], 0.)
    lg8 = dot(a2, w3_ref[...])
    ri = lax.broadcasted_iota(jnp.int32, lg8.shape, 0)
    li = lax.broadcasted_iota(jnp.int32, lg8.shape, 1)
    valid = ((ri % 7) < 6) | (li < 2)
    lgm = jnp.where(valid, lg8, -1e30)
    e8 = jnp.exp(lgm - jnp.max(lgm))
    s8 = dot(rep7t_ref[...], e8)
    sinv = 1. / jnp.sum(s8, axis=1, keepdims=True)
    att8 = e8 * dot(rep7_ref[...], sinv)
    p = o * dot(att8, t8)
    cs = dot(p, tsum_ref[...])
    out_ref[...] = dot(rep7t_ref[...], cs)


def _tc_compute(ep, hr8, uv_rep, w1, w2, wa1, wa2, w3, b1, b2, ba1, ba2,
                r1rt, t8, tsum, ttile, wa1b, rep7, rep7t):
    nblk = uv_rep.shape[0] // _BM
    full = lambda a: pl.BlockSpec(a.shape, lambda i: (0,) * a.ndim)
    return pl.pallas_call(
        _tc_body,
        grid=(nblk,),
        in_specs=[
            pl.BlockSpec((_RPB, 256), lambda i: (i, 0)),
            pl.BlockSpec((_RPB, 8), lambda i: (i, 0)),
            pl.BlockSpec((_BM, _D), lambda i: (i, 0)),
            full(w1), full(w2), full(wa1), full(wa2), full(w3),
            full(b1), full(b2), full(ba1), full(ba2), full(r1rt),
            full(t8), full(tsum), full(ttile), full(wa1b),
            full(rep7), full(rep7t),
        ],
        out_specs=pl.BlockSpec((_BM, _D), lambda i: (i, 0)),
        out_shape=jax.ShapeDtypeStruct((uv_rep.shape[0], _D), jnp.float32),
    )(ep, hr8, uv_rep, w1, w2, wa1, wa2, w3, b1, b2, ba1, ba2,
      r1rt, t8, tsum, ttile, wa1b, rep7, rep7t)


def kernel(nodes, history_uv, history_r, v2e, u2e, r2e, w_r1, b_r1, w_r2,
           b_r2, att_w1, att_b1, att_w2, att_b2, att_w3, att_b3):
    f32 = jnp.float32
    b, l = history_uv.shape
    d = v2e.shape[1]
    hist = jnp.pad(history_uv, ((0, 0), (0, _LP - l))).astype(jnp.int32)
    hr = jnp.pad(history_r, ((0, 0), (0, _LP - l)))
    hist2d4 = (hist.reshape(b * _LP // 128, 128) * 4).astype(jnp.int32)
    v2e_p = _transpose_pad_table(v2e)
    e_uv = _sc_gather(hist2d4, v2e_p.reshape(-1, 32))
    uv_rep = jnp.take(u2e, nodes, axis=0)
    ep = e_uv.reshape(b * _LP // _PK, _PK * d)
    hr8 = hr.reshape(b * _LP // _PK, _PK).astype(f32)
    eye8 = jnp.eye(_PK, dtype=f32)
    w1 = jnp.kron(eye8, w_r1[:d])
    w2 = jnp.kron(eye8, w_r2)
    wa1 = jnp.kron(eye8, att_w1[:d])
    wa2 = jnp.kron(eye8, att_w2)
    w3 = jnp.kron(eye8, att_w3)
    b1t = jnp.tile(b_r1, _PK)[None, :]
    b2t = jnp.tile(b_r2, _PK)[None, :]
    ba1t = jnp.tile(att_b1, _PK)[None, :]
    ba2t = jnp.tile(att_b2, _PK)[None, :]
    r1rt = jnp.tile(r2e @ w_r1[d:], (1, _PK))
    t8 = jnp.kron(eye8, jnp.ones((1, d), f32))
    tsum = jnp.kron(jnp.ones((_PK, 1), f32), jnp.eye(d, dtype=f32))
    ttile = tsum.T
    rep7 = (jnp.arange(_BM * _LP // _PK)[:, None] // (_LP // _PK)
            == jnp.arange(_BM)[None, :]).astype(f32)
    return _tc_compute(ep, hr8, uv_rep, w1, w2, wa1, wa2, w3,
                       b1t, b2t, ba1t, ba2t, r1rt, t8, tsum, ttile,
                       att_w1[d:], rep7, rep7.T)


# speedup vs baseline: 1.8134x; 1.8134x over previous
import functools

import jax
import jax.numpy as jnp
from jax import lax
from jax.experimental import pallas as pl
from jax.experimental.pallas import tpu as pltpu
from jax.experimental.pallas import tpu_sc as plsc

_D = 32
_LP = 56
_PK = 8
_BM = 128
_RPB = _BM * _LP // _PK

_NW = 32


def _transpose_pad_table(tab):
    tab_t = tab.T
    n = tab_t.shape[1]
    bw = 4096

    def body(x_ref, o_ref):
        y = x_ref[...].T
        o_ref[...] = jnp.concatenate(
            [y, jnp.zeros((y.shape[0], 96), jnp.float32)], axis=1)

    return pl.pallas_call(
        body,
        grid=(pl.cdiv(n, bw),),
        in_specs=[pl.BlockSpec((32, bw), lambda i: (0, i))],
        out_specs=pl.BlockSpec((bw, 128), lambda i: (i, 0)),
        out_shape=jax.ShapeDtypeStruct((n, 128), jnp.float32),
    )(tab_t)


def _sc_gather(hist2d4, v2e_rows):
    nrow = hist2d4.shape[0]
    rw = nrow // _NW
    d = 32
    mesh = plsc.VectorSubcoreMesh(core_axis_name="c", subcore_axis_name="s")
    kbuf = 2
    assert rw % kbuf == 0

    @functools.partial(
        pl.kernel,
        out_type=jax.ShapeDtypeStruct((nrow * 128, d), jnp.float32),
        mesh=mesh,
        scratch_types=[
            pltpu.VMEM((rw, 128), jnp.int32),
            [pltpu.VMEM((128, d), jnp.float32) for _ in range(kbuf)],
            [pltpu.SemaphoreType.DMA for _ in range(kbuf)],
            [pltpu.SemaphoreType.DMA for _ in range(kbuf)],
        ],
        compiler_params=pltpu.CompilerParams(use_tc_tiling_on_sc=False),
    )
    def k(hist_hbm, v2e_hbm, euv_out, idx_v, bufs, gsems, wsems):
        wid = lax.axis_index("s") * 2 + lax.axis_index("c")
        pltpu.sync_copy(hist_hbm.at[pl.ds(wid * rw, rw), :], idx_v)

        def group(g, carry):
            c0 = g * kbuf
            for b in range(kbuf):
                pltpu.async_copy(v2e_hbm.at[idx_v.at[c0 + b]], bufs[b],
                                 gsems[b])
            for b in range(kbuf):
                base = (wid * rw + c0 + b) * 128
                pltpu.make_async_copy(v2e_hbm.at[idx_v.at[c0 + b]], bufs[b],
                                      gsems[b]).wait()
                pltpu.async_copy(bufs[b], euv_out.at[pl.ds(base, 128), :],
                                 wsems[b])
            for b in range(kbuf):
                base = (wid * rw + c0 + b) * 128
                pltpu.make_async_copy(bufs[b], euv_out.at[pl.ds(base, 128), :],
                                      wsems[b]).wait()
            return carry

        lax.fori_loop(0, rw // kbuf, group, 0)

    return k(hist2d4, v2e_rows)


def _tc_body(ep_ref, hr_ref, uv_ref, w1_ref, w2_ref, wa1_ref, wa2_ref,
             w3_ref, b1_ref, b2_ref, ba1_ref, ba2_ref, r1rt_ref, t8_ref,
             tsum_ref, ttile_ref, wa1b_ref, rep7_ref, rep7t_ref, out_ref):
    f32 = jnp.float32
    dot = functools.partial(jnp.dot, preferred_element_type=f32)
    e = ep_ref[...].astype(f32)
    t8 = t8_ref[...]
    hre = dot(hr_ref[...], t8)
    r1rt = r1rt_ref[...]
    er = jnp.where(hre == 0., r1rt[0:1],
         jnp.where(hre == 1., r1rt[1:2],
         jnp.where(hre == 2., r1rt[2:3],
         jnp.where(hre == 3., r1rt[3:4], r1rt[4:5]))))
    x = jnp.maximum(dot(e, w1_ref[...]) + er + b1_ref[...], 0.)
    o = jnp.maximum(dot(x, w2_ref[...]) + b2_ref[...], 0.)
    uvp = dot(dot(uv_ref[...], wa1b_ref[...]), ttile_ref[...])
    uvr = dot(rep7_ref[...], uvp)
    a1 = jnp.maximum(dot(o, wa1_ref[...]) + uvr + ba1_ref[...], 0.)
    a2 = jnp.maximum(dot(a1, wa2_ref[...]) + ba2_ref[...], 0.)
    lg8 = dot(a2, w3_ref[...])
    ri = lax.broadcasted_iota(jnp.int32, lg8.shape, 0)
    li = lax.broadcasted_iota(jnp.int32, lg8.shape, 1)
    valid = ((ri % 7) < 6) | (li < 2)
    lgm = jnp.where(valid, lg8, -1e30)
    e8 = jnp.exp(lgm - jnp.max(lgm))
    s8 = dot(rep7t_ref[...], e8)
    sinv = 1. / jnp.sum(s8, axis=1, keepdims=True)
    att8 = e8 * dot(rep7_ref[...], sinv)
    p = o * dot(att8, t8)
    cs = dot(p, tsum_ref[...])
    out_ref[...] = dot(rep7t_ref[...], cs)


def _tc_compute(ep, hr8, uv_rep, w1, w2, wa1, wa2, w3, b1, b2, ba1, ba2,
                r1rt, t8, tsum, ttile, wa1b, rep7, rep7t):
    nblk = uv_rep.shape[0] // _BM
    full = lambda a: pl.BlockSpec(a.shape, lambda i: (0,) * a.ndim)
    return pl.pallas_call(
        _tc_body,
        grid=(nblk,),
        in_specs=[
            pl.BlockSpec((_RPB, 256), lambda i: (i, 0)),
            pl.BlockSpec((_RPB, 8), lambda i: (i, 0)),
            pl.BlockSpec((_BM, _D), lambda i: (i, 0)),
            full(w1), full(w2), full(wa1), full(wa2), full(w3),
            full(b1), full(b2), full(ba1), full(ba2), full(r1rt),
            full(t8), full(tsum), full(ttile), full(wa1b),
            full(rep7), full(rep7t),
        ],
        out_specs=pl.BlockSpec((_BM, _D), lambda i: (i, 0)),
        out_shape=jax.ShapeDtypeStruct((uv_rep.shape[0], _D), jnp.float32),
    )(ep, hr8, uv_rep, w1, w2, wa1, wa2, w3, b1, b2, ba1, ba2,
      r1rt, t8, tsum, ttile, wa1b, rep7, rep7t)


def kernel(nodes, history_uv, history_r, v2e, u2e, r2e, w_r1, b_r1, w_r2,
           b_r2, att_w1, att_b1, att_w2, att_b2, att_w3, att_b3):
    f32 = jnp.float32
    b, l = history_uv.shape
    d = v2e.shape[1]
    hist = jnp.pad(history_uv, ((0, 0), (0, _LP - l))).astype(jnp.int32)
    hr = jnp.pad(history_r, ((0, 0), (0, _LP - l)))
    hist2d4 = (hist.reshape(b * _LP // 128, 128) * 4).astype(jnp.int32)
    v2e_p = _transpose_pad_table(v2e)
    e_uv = _sc_gather(hist2d4, v2e_p.reshape(-1, 32))
    uv_rep = jnp.take(u2e, nodes, axis=0)
    ep = e_uv.reshape(b * _LP // _PK, _PK * d)
    hr8 = hr.reshape(b * _LP // _PK, _PK).astype(f32)
    eye8 = jnp.eye(_PK, dtype=f32)
    w1 = jnp.kron(eye8, w_r1[:d])
    w2 = jnp.kron(eye8, w_r2)
    wa1 = jnp.kron(eye8, att_w1[:d])
    wa2 = jnp.kron(eye8, att_w2)
    w3 = jnp.kron(eye8, att_w3)
    b1t = jnp.tile(b_r1, _PK)[None, :]
    b2t = jnp.tile(b_r2, _PK)[None, :]
    ba1t = jnp.tile(att_b1, _PK)[None, :]
    ba2t = jnp.tile(att_b2, _PK)[None, :]
    r1rt = jnp.tile(r2e @ w_r1[d:], (1, _PK))
    t8 = jnp.kron(eye8, jnp.ones((1, d), f32))
    tsum = jnp.kron(jnp.ones((_PK, 1), f32), jnp.eye(d, dtype=f32))
    ttile = tsum.T
    rep7 = (jnp.arange(_BM * _LP // _PK)[:, None] // (_LP // _PK)
            == jnp.arange(_BM)[None, :]).astype(f32)
    return _tc_compute(ep, hr8, uv_rep, w1, w2, wa1, wa2, w3,
                       b1t, b2t, ba1t, ba2t, r1rt, t8, tsum, ttile,
                       att_w1[d:], rep7, rep7.T)


# speedup vs baseline: 1.8332x; 1.0109x over previous
import functools

import jax
import jax.numpy as jnp
from jax import lax
from jax.experimental import pallas as pl
from jax.experimental.pallas import tpu as pltpu
from jax.experimental.pallas import tpu_sc as plsc

_D = 32
_LP = 56
_PK = 8
_BM = 128
_RPB = _BM * _LP // _PK

_NW = 32


def _transpose_pad_table(tab):
    tab_t = tab.T
    n = tab_t.shape[1]
    bw = 4096

    def body(x_ref, o_ref):
        y = x_ref[...].T
        o_ref[...] = jnp.concatenate(
            [y, jnp.zeros((y.shape[0], 96), jnp.float32)], axis=1)

    return pl.pallas_call(
        body,
        grid=(pl.cdiv(n, bw),),
        in_specs=[pl.BlockSpec((32, bw), lambda i: (0, i))],
        out_specs=pl.BlockSpec((bw, 128), lambda i: (i, 0)),
        out_shape=jax.ShapeDtypeStruct((n, 128), jnp.float32),
    )(tab_t)


def _sc_gather(hist2d4, v2e_rows):
    nrow = hist2d4.shape[0]
    rw = nrow // _NW
    d = 32
    mesh = plsc.VectorSubcoreMesh(core_axis_name="c", subcore_axis_name="s")
    kbuf = 2
    assert rw % kbuf == 0

    @functools.partial(
        pl.kernel,
        out_type=jax.ShapeDtypeStruct((nrow * 128, d), jnp.float32),
        mesh=mesh,
        scratch_types=[
            pltpu.VMEM((rw, 128), jnp.int32),
            [pltpu.VMEM((128, d), jnp.float32) for _ in range(kbuf)],
            [pltpu.SemaphoreType.DMA for _ in range(kbuf)],
            [pltpu.SemaphoreType.DMA for _ in range(kbuf)],
        ],
        compiler_params=pltpu.CompilerParams(use_tc_tiling_on_sc=False),
    )
    def k(hist_hbm, v2e_hbm, euv_out, idx_v, bufs, gsems, wsems):
        wid = lax.axis_index("s") * 2 + lax.axis_index("c")
        pltpu.sync_copy(hist_hbm.at[pl.ds(wid * rw, rw), :], idx_v)

        def group(g, carry):
            c0 = g * kbuf
            for b in range(kbuf):
                pltpu.async_copy(v2e_hbm.at[idx_v.at[c0 + b]], bufs[b],
                                 gsems[b])
            for b in range(kbuf):
                base = (wid * rw + c0 + b) * 128
                pltpu.make_async_copy(v2e_hbm.at[idx_v.at[c0 + b]], bufs[b],
                                      gsems[b]).wait()
                pltpu.async_copy(bufs[b], euv_out.at[pl.ds(base, 128), :],
                                 wsems[b])
            for b in range(kbuf):
                base = (wid * rw + c0 + b) * 128
                pltpu.make_async_copy(bufs[b], euv_out.at[pl.ds(base, 128), :],
                                      wsems[b]).wait()
            return carry

        lax.fori_loop(0, rw // kbuf, group, 0)

    return k(hist2d4, v2e_rows)


def _tc_body(ep_ref, hr_ref, uv_ref, w1_ref, w2_ref, wa1_ref, wa2_ref,
             w3_ref, b1_ref, b2_ref, ba1_ref, ba2_ref, r1rt_ref, t8_ref,
             tsum_ref, ttile_ref, wa1b_ref, rep7_ref, rep7t_ref, out_ref):
    f32 = jnp.float32
    dot = functools.partial(jnp.dot, preferred_element_type=f32)
    e = ep_ref[...].astype(f32)
    t8 = t8_ref[...]
    hre = dot(hr_ref[...], t8)
    r1rt = r1rt_ref[...]
    er = jnp.where(hre == 0., r1rt[0:1],
         jnp.where(hre == 1., r1rt[1:2],
         jnp.where(hre == 2., r1rt[2:3],
         jnp.where(hre == 3., r1rt[3:4], r1rt[4:5]))))
    bf16 = jnp.bfloat16
    x = jnp.maximum(dot(e.astype(bf16), w1_ref[...]) + er + b1_ref[...], 0.)
    o = jnp.maximum(dot(x.astype(bf16), w2_ref[...]) + b2_ref[...], 0.)
    uvp = dot(dot(uv_ref[...], wa1b_ref[...]), ttile_ref[...])
    uvr = dot(rep7_ref[...], uvp)
    a1 = jnp.maximum(dot(o.astype(bf16), wa1_ref[...]) + uvr + ba1_ref[...],
                     0.)
    a2 = jnp.maximum(dot(a1.astype(bf16), wa2_ref[...]) + ba2_ref[...], 0.)
    lg8 = dot(a2, w3_ref[...])
    ri = lax.broadcasted_iota(jnp.int32, lg8.shape, 0)
    li = lax.broadcasted_iota(jnp.int32, lg8.shape, 1)
    valid = ((ri % 7) < 6) | (li < 2)
    lgm = jnp.where(valid, lg8, -1e30)
    e8 = jnp.exp(lgm - jnp.max(lgm))
    s8 = dot(rep7t_ref[...], e8)
    sinv = 1. / jnp.sum(s8, axis=1, keepdims=True)
    att8 = e8 * dot(rep7_ref[...], sinv)
    p = o * dot(att8, t8)
    cs = dot(p, tsum_ref[...])
    out_ref[...] = dot(rep7t_ref[...], cs)


def _tc_compute(ep, hr8, uv_rep, w1, w2, wa1, wa2, w3, b1, b2, ba1, ba2,
                r1rt, t8, tsum, ttile, wa1b, rep7, rep7t):
    nblk = uv_rep.shape[0] // _BM
    full = lambda a: pl.BlockSpec(a.shape, lambda i: (0,) * a.ndim)
    return pl.pallas_call(
        _tc_body,
        grid=(nblk,),
        in_specs=[
            pl.BlockSpec((_RPB, 256), lambda i: (i, 0)),
            pl.BlockSpec((_RPB, 8), lambda i: (i, 0)),
            pl.BlockSpec((_BM, _D), lambda i: (i, 0)),
            full(w1), full(w2), full(wa1), full(wa2), full(w3),
            full(b1), full(b2), full(ba1), full(ba2), full(r1rt),
            full(t8), full(tsum), full(ttile), full(wa1b),
            full(rep7), full(rep7t),
        ],
        out_specs=pl.BlockSpec((_BM, _D), lambda i: (i, 0)),
        out_shape=jax.ShapeDtypeStruct((uv_rep.shape[0], _D), jnp.float32),
    )(ep, hr8, uv_rep, w1, w2, wa1, wa2, w3, b1, b2, ba1, ba2,
      r1rt, t8, tsum, ttile, wa1b, rep7, rep7t)


def kernel(nodes, history_uv, history_r, v2e, u2e, r2e, w_r1, b_r1, w_r2,
           b_r2, att_w1, att_b1, att_w2, att_b2, att_w3, att_b3):
    f32 = jnp.float32
    b, l = history_uv.shape
    d = v2e.shape[1]
    hist = jnp.pad(history_uv, ((0, 0), (0, _LP - l))).astype(jnp.int32)
    hr = jnp.pad(history_r, ((0, 0), (0, _LP - l)))
    hist2d4 = (hist.reshape(b * _LP // 128, 128) * 4).astype(jnp.int32)
    v2e_p = _transpose_pad_table(v2e)
    e_uv = _sc_gather(hist2d4, v2e_p.reshape(-1, 32))
    uv_rep = jnp.take(u2e, nodes, axis=0)
    ep = e_uv.reshape(b * _LP // _PK, _PK * d)
    hr8 = hr.reshape(b * _LP // _PK, _PK).astype(f32)
    eye8 = jnp.eye(_PK, dtype=f32)
    w1 = jnp.kron(eye8, w_r1[:d]).astype(jnp.bfloat16)
    w2 = jnp.kron(eye8, w_r2).astype(jnp.bfloat16)
    wa1 = jnp.kron(eye8, att_w1[:d]).astype(jnp.bfloat16)
    wa2 = jnp.kron(eye8, att_w2).astype(jnp.bfloat16)
    w3 = jnp.kron(eye8, att_w3)
    b1t = jnp.tile(b_r1, _PK)[None, :]
    b2t = jnp.tile(b_r2, _PK)[None, :]
    ba1t = jnp.tile(att_b1, _PK)[None, :]
    ba2t = jnp.tile(att_b2, _PK)[None, :]
    r1rt = jnp.tile(r2e @ w_r1[d:], (1, _PK))
    t8 = jnp.kron(eye8, jnp.ones((1, d), f32))
    tsum = jnp.kron(jnp.ones((_PK, 1), f32), jnp.eye(d, dtype=f32))
    ttile = tsum.T
    rep7 = (jnp.arange(_BM * _LP // _PK)[:, None] // (_LP // _PK)
            == jnp.arange(_BM)[None, :]).astype(f32)
    return _tc_compute(ep, hr8, uv_rep, w1, w2, wa1, wa2, w3,
                       b1t, b2t, ba1t, ba2t, r1rt, t8, tsum, ttile,
                       att_w1[d:], rep7, rep7.T)
